# baseline (speedup 1.0000x reference)
import jax
import jax.numpy as jnp
from jax.experimental import pallas as pl


def _add_kernel(x_ref, e_ref, o_ref):
    o_ref[...] = x_ref[...] + e_ref[...]


def kernel(x, embed):
    B, S, D = x.shape
    e = embed[:S]
    BLK = 256
    grid = (S // BLK, B)
    return pl.pallas_call(
        _add_kernel,
        grid=grid,
        in_specs=[
            pl.BlockSpec((1, BLK, D), lambda i, b: (b, i, 0)),
            pl.BlockSpec((BLK, D), lambda i, b: (i, 0)),
        ],
        out_specs=pl.BlockSpec((1, BLK, D), lambda i, b: (b, i, 0)),
        out_shape=jax.ShapeDtypeStruct(x.shape, x.dtype),
    )(x, e)


# speedup vs baseline: 1.1594x; 1.1594x over previous
import jax
import jax.numpy as jnp
from jax.experimental import pallas as pl


def _add_kernel(x_ref, e_ref, o_ref):
    o_ref[...] = x_ref[...] + e_ref[...]


def kernel(x, embed):
    B, S, D = x.shape
    e = embed[:S]
    BLK = 256
    grid = (S // BLK,)
    return pl.pallas_call(
        _add_kernel,
        grid=grid,
        in_specs=[
            pl.BlockSpec((B, BLK, D), lambda i: (0, i, 0)),
            pl.BlockSpec((BLK, D), lambda i: (i, 0)),
        ],
        out_specs=pl.BlockSpec((B, BLK, D), lambda i: (0, i, 0)),
        out_shape=jax.ShapeDtypeStruct(x.shape, x.dtype),
    )(x, e)
